# baseline (device time: 140752 ns/iter reference)
import jax
import jax.numpy as jnp
from jax import lax
from jax.experimental import pallas as pl
from jax.experimental.pallas import tpu as pltpu

N_DEV = 32
CAP = 6
PAD = 8


def kernel(x, router_W, route_idx, expert_W):
    n_tok, d = x.shape
    n_loc, _, h = expert_W.shape
    E = N_DEV * n_loc
    S = n_loc * PAD

    e = route_idx[:, 0].astype(jnp.int32)
    onehot = (e[:, None] == jnp.arange(E, dtype=jnp.int32)[None, :]).astype(
        jnp.int32
    )
    rank = (
        jnp.take_along_axis(jnp.cumsum(onehot, axis=0), e[:, None], axis=1)[:, 0]
        - 1
    )
    keep = rank < CAP
    slot = e * PAD + rank
    dest = jnp.full((E * PAD,), -1, dtype=jnp.int32)
    dest = dest.at[jnp.where(keep, slot, E * PAD)].set(
        jnp.arange(n_tok, dtype=jnp.int32), mode="drop"
    )
    dest_tok = dest.reshape(N_DEV, S)

    my = lax.axis_index("i")
    my_dest = lax.dynamic_index_in_dim(dest_tok, my, 0, keepdims=False)
    compact_x = jnp.where(
        (my_dest >= 0)[:, None],
        x[jnp.clip(my_dest, 0, n_tok - 1)],
        jnp.zeros((), jnp.float32),
    )

    def body(cx_ref, ew_ref, dest_ref, out_ref, comm_ref, send_sems, recv_sems):
        my_pos = lax.axis_index("i")
        left = lax.rem(my_pos + N_DEV - 1, N_DEV)
        right = lax.rem(my_pos + 1, N_DEV)

        barrier_sem = pltpu.get_barrier_semaphore()
        for nbr in (left, right):
            pl.semaphore_signal(
                barrier_sem,
                inc=1,
                device_id=(nbr,),
                device_id_type=pl.DeviceIdType.MESH,
            )
        pl.semaphore_wait(barrier_sem, 2)

        out_ref[:, :] = jnp.zeros((n_tok, h), jnp.float32)

        for k in range(n_loc):
            r = lax.dot_general(
                cx_ref[PAD * k : PAD * (k + 1), :],
                ew_ref[k],
                (((1,), (0,)), ((), ())),
                preferred_element_type=jnp.float32,
            )
            comm_ref[pl.ds(my_pos, 1), pl.ds(PAD * k, PAD), :] = r[None]

        def scatter_chunk(s):
            def body_j(j, carry):
                dtok = dest_ref[s, j]

                @pl.when(dtok >= 0)
                def _():
                    out_ref[pl.ds(dtok, 1), :] = comm_ref[
                        pl.ds(s, 1), pl.ds(j, 1), :
                    ][0]

                return carry

            lax.fori_loop(0, S, body_j, 0)

        scatter_chunk(my_pos)

        for t in range(N_DEV - 1):
            o_send = lax.rem(my_pos - t + N_DEV, N_DEV)
            rdma = pltpu.make_async_remote_copy(
                src_ref=comm_ref.at[pl.ds(o_send, 1)],
                dst_ref=comm_ref.at[pl.ds(o_send, 1)],
                send_sem=send_sems.at[t],
                recv_sem=recv_sems.at[t],
                device_id=(right,),
                device_id_type=pl.DeviceIdType.MESH,
            )
            rdma.start()
            rdma.wait()
            o_recv = lax.rem(my_pos - t - 1 + N_DEV, N_DEV)
            scatter_chunk(o_recv)

    return pl.pallas_call(
        body,
        out_shape=jax.ShapeDtypeStruct((n_tok, h), jnp.float32),
        in_specs=[
            pl.BlockSpec(memory_space=pltpu.VMEM),
            pl.BlockSpec(memory_space=pltpu.VMEM),
            pl.BlockSpec(memory_space=pltpu.SMEM),
        ],
        out_specs=pl.BlockSpec(memory_space=pltpu.VMEM),
        scratch_shapes=[
            pltpu.VMEM((N_DEV, S, h), jnp.float32),
            pltpu.SemaphoreType.DMA((N_DEV - 1,)),
            pltpu.SemaphoreType.DMA((N_DEV - 1,)),
        ],
        compiler_params=pltpu.CompilerParams(collective_id=0),
    )(compact_x, expert_W, dest_tok)


# device time: 71115 ns/iter; 1.9792x vs baseline; 1.9792x over previous
import jax
import jax.numpy as jnp
from jax import lax
from jax.experimental import pallas as pl
from jax.experimental.pallas import tpu as pltpu

N_DEV = 32
CAP = 6


def kernel(x, router_W, route_idx, expert_W):
    n_tok, d = x.shape
    n_loc, _, h = expert_W.shape
    E = N_DEV * n_loc
    S = n_loc * CAP

    e = route_idx[:, 0].astype(jnp.int32)
    onehot = (e[:, None] == jnp.arange(E, dtype=jnp.int32)[None, :]).astype(
        jnp.int32
    )
    rank = (
        jnp.take_along_axis(jnp.cumsum(onehot, axis=0), e[:, None], axis=1)[:, 0]
        - 1
    )
    keep = rank < CAP
    slot = e * CAP + rank
    dest = jnp.full((E * CAP,), -1, dtype=jnp.int32)
    dest = dest.at[jnp.where(keep, slot, E * CAP)].set(
        jnp.arange(n_tok, dtype=jnp.int32), mode="drop"
    )
    dest_tok = dest.reshape(N_DEV, S)

    my = lax.axis_index("i")
    my_dest = lax.dynamic_index_in_dim(dest_tok, my, 0, keepdims=False)
    compact_x = jnp.where(
        (my_dest >= 0)[:, None],
        x[jnp.clip(my_dest, 0, n_tok - 1)],
        jnp.zeros((), jnp.float32),
    )

    def body(cx_ref, ew_ref, dest_ref, out_ref, comm_ref, send_sems, recv_sems):
        my_pos = lax.axis_index("i")

        barrier_sem = pltpu.get_barrier_semaphore()
        for off in range(1, N_DEV):
            pl.semaphore_signal(
                barrier_sem,
                inc=1,
                device_id=(lax.rem(my_pos + off, N_DEV),),
                device_id_type=pl.DeviceIdType.MESH,
            )
        pl.semaphore_wait(barrier_sem, N_DEV - 1)

        for k in range(n_loc):
            r = lax.dot_general(
                cx_ref[CAP * k : CAP * (k + 1), :],
                ew_ref[k],
                (((1,), (0,)), ((), ())),
                preferred_element_type=jnp.float32,
            )
            comm_ref[pl.ds(my_pos, 1), pl.ds(CAP * k, CAP), :] = r[None]

        def send_to(tgt):
            return pltpu.make_async_remote_copy(
                src_ref=comm_ref.at[pl.ds(my_pos, 1)],
                dst_ref=comm_ref.at[pl.ds(my_pos, 1)],
                send_sem=send_sems.at[tgt],
                recv_sem=recv_sems.at[my_pos],
                device_id=(tgt,),
                device_id_type=pl.DeviceIdType.MESH,
            )

        def recv_from(origin):
            return pltpu.make_async_remote_copy(
                src_ref=comm_ref.at[pl.ds(origin, 1)],
                dst_ref=comm_ref.at[pl.ds(origin, 1)],
                send_sem=send_sems.at[origin],
                recv_sem=recv_sems.at[origin],
                device_id=(origin,),
                device_id_type=pl.DeviceIdType.MESH,
            )

        for off in range(1, N_DEV):
            send_to(lax.rem(my_pos + off, N_DEV)).start()

        out_ref[:, :] = jnp.zeros((n_tok, h), jnp.float32)

        def scatter_chunk(s):
            def body_j(j, carry):
                dtok = dest_ref[s, j]

                @pl.when(dtok >= 0)
                def _():
                    out_ref[pl.ds(dtok, 1), :] = comm_ref[
                        pl.ds(s, 1), pl.ds(j, 1), :
                    ][0]

                return carry

            lax.fori_loop(0, S, body_j, 0)

        scatter_chunk(my_pos)

        for off in range(1, N_DEV):
            origin = lax.rem(my_pos - off + N_DEV, N_DEV)
            recv_from(origin).wait_recv()
            scatter_chunk(origin)

        for off in range(1, N_DEV):
            send_to(lax.rem(my_pos + off, N_DEV)).wait_send()

    return pl.pallas_call(
        body,
        out_shape=jax.ShapeDtypeStruct((n_tok, h), jnp.float32),
        in_specs=[
            pl.BlockSpec(memory_space=pltpu.VMEM),
            pl.BlockSpec(memory_space=pltpu.VMEM),
            pl.BlockSpec(memory_space=pltpu.SMEM),
        ],
        out_specs=pl.BlockSpec(memory_space=pltpu.VMEM),
        scratch_shapes=[
            pltpu.VMEM((N_DEV, S, h), jnp.float32),
            pltpu.SemaphoreType.DMA((N_DEV,)),
            pltpu.SemaphoreType.DMA((N_DEV,)),
        ],
        compiler_params=pltpu.CompilerParams(collective_id=0),
    )(compact_x, expert_W, dest_tok)


# device time: 53105 ns/iter; 2.6504x vs baseline; 1.3391x over previous
import jax
import jax.numpy as jnp
from jax import lax
from jax.experimental import pallas as pl
from jax.experimental.pallas import tpu as pltpu

N_DEV = 32
CAP = 6


def kernel(x, router_W, route_idx, expert_W):
    n_tok, d = x.shape
    n_loc, _, h = expert_W.shape
    E = N_DEV * n_loc
    S = n_loc * CAP

    e = route_idx[:, 0].astype(jnp.int32)
    onehot = (e[:, None] == jnp.arange(E, dtype=jnp.int32)[None, :]).astype(
        jnp.int32
    )
    rank = (
        jnp.take_along_axis(jnp.cumsum(onehot, axis=0), e[:, None], axis=1)[:, 0]
        - 1
    )
    keep = rank < CAP
    slot = e * CAP + rank
    dest = jnp.full((E * CAP,), -1, dtype=jnp.int32)
    dest = dest.at[jnp.where(keep, slot, E * CAP)].set(
        jnp.arange(n_tok, dtype=jnp.int32), mode="drop"
    )
    dest_tok = dest.reshape(N_DEV, S)

    def body(x_ref, ew_ref, dest_ref, out_ref, comm_ref, cx_ref, send_sems,
             recv_sems):
        my_pos = lax.axis_index("i")

        barrier_sem = pltpu.get_barrier_semaphore()
        for off in range(1, N_DEV):
            pl.semaphore_signal(
                barrier_sem,
                inc=1,
                device_id=(lax.rem(my_pos + off, N_DEV),),
                device_id_type=pl.DeviceIdType.MESH,
            )
        pl.semaphore_wait(barrier_sem, N_DEV - 1)

        cx_ref[:, :] = jnp.zeros((S, d), jnp.float32)
        for j in range(S):
            tok = dest_ref[my_pos, j]

            @pl.when(tok >= 0)
            def _():
                cx_ref[pl.ds(j, 1), :] = x_ref[pl.ds(tok, 1), :]

        for k in range(n_loc):
            r = lax.dot_general(
                cx_ref[CAP * k : CAP * (k + 1), :],
                ew_ref[k],
                (((1,), (0,)), ((), ())),
                preferred_element_type=jnp.float32,
            )
            comm_ref[pl.ds(my_pos, 1), pl.ds(CAP * k, CAP), :] = r.astype(
                jnp.bfloat16
            )[None]

        def send_to(tgt):
            return pltpu.make_async_remote_copy(
                src_ref=comm_ref.at[pl.ds(my_pos, 1)],
                dst_ref=comm_ref.at[pl.ds(my_pos, 1)],
                send_sem=send_sems.at[tgt],
                recv_sem=recv_sems.at[my_pos],
                device_id=(tgt,),
                device_id_type=pl.DeviceIdType.MESH,
            )

        def recv_from(origin):
            return pltpu.make_async_remote_copy(
                src_ref=comm_ref.at[pl.ds(origin, 1)],
                dst_ref=comm_ref.at[pl.ds(origin, 1)],
                send_sem=send_sems.at[origin],
                recv_sem=recv_sems.at[origin],
                device_id=(origin,),
                device_id_type=pl.DeviceIdType.MESH,
            )

        for off in range(1, N_DEV):
            send_to(lax.rem(my_pos + off, N_DEV)).start()

        out_ref[:, :] = jnp.zeros((n_tok, h), jnp.float32)

        def scatter_chunk(s):
            for j in range(S):
                dtok = dest_ref[s, j]

                @pl.when(dtok >= 0)
                def _():
                    out_ref[pl.ds(dtok, 1), :] = comm_ref[
                        pl.ds(s, 1), j, :
                    ].astype(jnp.float32)

        scatter_chunk(my_pos)

        for off in range(1, N_DEV):
            origin = lax.rem(my_pos - off + N_DEV, N_DEV)
            recv_from(origin).wait_recv()
            scatter_chunk(origin)

        for off in range(1, N_DEV):
            send_to(lax.rem(my_pos + off, N_DEV)).wait_send()

    return pl.pallas_call(
        body,
        out_shape=jax.ShapeDtypeStruct((n_tok, h), jnp.float32),
        in_specs=[
            pl.BlockSpec(memory_space=pltpu.VMEM),
            pl.BlockSpec(memory_space=pltpu.VMEM),
            pl.BlockSpec(memory_space=pltpu.SMEM),
        ],
        out_specs=pl.BlockSpec(memory_space=pltpu.VMEM),
        scratch_shapes=[
            pltpu.VMEM((N_DEV, S, h), jnp.bfloat16),
            pltpu.VMEM((S, d), jnp.float32),
            pltpu.SemaphoreType.DMA((N_DEV,)),
            pltpu.SemaphoreType.DMA((N_DEV,)),
        ],
        compiler_params=pltpu.CompilerParams(collective_id=0),
    )(x, expert_W, dest_tok)


# device time: 44774 ns/iter; 3.1436x vs baseline; 1.1861x over previous
import jax
import jax.numpy as jnp
from jax import lax
from jax.experimental import pallas as pl
from jax.experimental.pallas import tpu as pltpu

N_DEV = 32
CAP = 6


def kernel(x, router_W, route_idx, expert_W):
    n_tok, d = x.shape
    n_loc, _, h = expert_W.shape
    E = N_DEV * n_loc
    S = n_loc * CAP

    e = route_idx[:, 0].astype(jnp.int32)
    onehot = (e[:, None] == jnp.arange(E, dtype=jnp.int32)[None, :]).astype(
        jnp.int32
    )
    rank = jnp.sum(jnp.cumsum(onehot, axis=0) * onehot, axis=1) - 1
    keep = rank < CAP
    slot = e * CAP + rank
    dest = jnp.full((E * CAP,), -1, dtype=jnp.int32)
    dest = dest.at[jnp.where(keep, slot, E * CAP)].set(
        jnp.arange(n_tok, dtype=jnp.int32), mode="drop"
    )
    dest_tok = dest.reshape(N_DEV, S)

    def body(x_ref, ew_ref, dest_ref, out_ref, comm_ref, cx_ref, stage_ref,
             send_sems, recv_sems):
        my_pos = lax.axis_index("i")

        barrier_sem = pltpu.get_barrier_semaphore()
        for off in range(1, N_DEV):
            pl.semaphore_signal(
                barrier_sem,
                inc=1,
                device_id=(lax.rem(my_pos + off, N_DEV),),
                device_id_type=pl.DeviceIdType.MESH,
            )
        pl.semaphore_wait(barrier_sem, N_DEV - 1)

        cx_ref[:, :] = jnp.zeros((S, d), jnp.float32)
        for j in range(S):
            tok = dest_ref[my_pos, j]

            @pl.when(tok >= 0)
            def _():
                cx_ref[pl.ds(j, 1), :] = x_ref[pl.ds(tok, 1), :]

        for k in range(n_loc):
            r = lax.dot_general(
                cx_ref[CAP * k : CAP * (k + 1), :],
                ew_ref[k],
                (((1,), (0,)), ((), ())),
                preferred_element_type=jnp.float32,
            )
            comm_ref[pl.ds(my_pos, 1), pl.ds(CAP * k, CAP), :] = r.astype(
                jnp.bfloat16
            )[None]

        def send_to(tgt):
            return pltpu.make_async_remote_copy(
                src_ref=comm_ref.at[pl.ds(my_pos, 1)],
                dst_ref=comm_ref.at[pl.ds(my_pos, 1)],
                send_sem=send_sems.at[tgt],
                recv_sem=recv_sems.at[my_pos],
                device_id=(tgt,),
                device_id_type=pl.DeviceIdType.MESH,
            )

        def recv_from(origin):
            return pltpu.make_async_remote_copy(
                src_ref=comm_ref.at[pl.ds(origin, 1)],
                dst_ref=comm_ref.at[pl.ds(origin, 1)],
                send_sem=send_sems.at[origin],
                recv_sem=recv_sems.at[origin],
                device_id=(origin,),
                device_id_type=pl.DeviceIdType.MESH,
            )

        for off in range(1, N_DEV):
            send_to(lax.rem(my_pos + off, N_DEV)).start()

        out_ref[:, :] = jnp.zeros((n_tok, h), jnp.float32)

        def scatter_chunk(s):
            stage_ref[:, :] = comm_ref[pl.ds(s, 1)][0].astype(jnp.float32)
            for j in range(S):
                dtok = dest_ref[s, j]

                @pl.when(dtok >= 0)
                def _():
                    out_ref[pl.ds(dtok, 1), :] = stage_ref[pl.ds(j, 1), :]

        scatter_chunk(my_pos)

        for off in range(1, N_DEV):
            origin = lax.rem(my_pos - off + N_DEV, N_DEV)
            recv_from(origin).wait_recv()
            scatter_chunk(origin)

        for off in range(1, N_DEV):
            send_to(lax.rem(my_pos + off, N_DEV)).wait_send()

    return pl.pallas_call(
        body,
        out_shape=jax.ShapeDtypeStruct((n_tok, h), jnp.float32),
        in_specs=[
            pl.BlockSpec(memory_space=pltpu.VMEM),
            pl.BlockSpec(memory_space=pltpu.VMEM),
            pl.BlockSpec(memory_space=pltpu.SMEM),
        ],
        out_specs=pl.BlockSpec(memory_space=pltpu.VMEM),
        scratch_shapes=[
            pltpu.VMEM((N_DEV, S, h), jnp.bfloat16),
            pltpu.VMEM((S, d), jnp.float32),
            pltpu.VMEM((S, h), jnp.float32),
            pltpu.SemaphoreType.DMA((N_DEV,)),
            pltpu.SemaphoreType.DMA((N_DEV,)),
        ],
        compiler_params=pltpu.CompilerParams(collective_id=0),
    )(x, expert_W, dest_tok)
